# baseline (speedup 1.0000x reference)
import dataclasses
import functools

import jax
import jax.numpy as jnp
from jax import lax
from jax.experimental import pallas as pl
from jax.experimental.pallas import tpu as pltpu
from jax.experimental.pallas import tpu_sc as plsc

N_NODES = 100000
D = 128
E_EDGES = 200000

NC = 2
NS = 16
BIN = 12288
PASS_SPAN = NC * BIN
NPASS = -(-N_NODES // PASS_SPAN)
AGG_ROWS = NPASS * PASS_SPAN
DUMMY = BIN
SPMEM_ROWS = BIN + 128
SLICE = 12800
EP = NS * SLICE
CH = 128
EC = 3200
NEC = SLICE // EC
SELROWS = EC // CH + 1
NSUB = BIN // NS
PAD_DST = 1 << 20

DEG_ROWS = N_NODES + 352
DSUB = DEG_ROWS // NS
DCH = 56
EP_DEG = NC * NS * DCH * CH

_SC_PARAMS = dataclasses.replace(
    pltpu.CompilerParams(), needs_layout_passes=False
)


def _sc_aggregate(feat, src, dst):
    mesh = plsc.VectorSubcoreMesh(core_axis_name="c", subcore_axis_name="s")

    @functools.partial(
        pl.kernel,
        out_type=jax.ShapeDtypeStruct((AGG_ROWS, D), jnp.float32),
        mesh=mesh,
        scratch_types=[
            pltpu.VMEM((EC,), jnp.int32),
            pltpu.VMEM((EC,), jnp.int32),
            pltpu.VMEM((SELROWS, CH), jnp.int32),
            pltpu.VMEM((SELROWS, CH), jnp.int32),
            pltpu.VMEM((CH, D), jnp.float32),
            pltpu.VMEM_SHARED((SPMEM_ROWS, D), jnp.float32),
        ],
        compiler_params=_SC_PARAMS,
    )
    def k(feat_hbm, src_hbm, dst_hbm, agg_hbm,
          src_c, dst_c, sel_src, sel_dst, grows, sp_agg):
        c = lax.axis_index("c")
        s = lax.axis_index("s")
        zero16 = jnp.zeros((16,), jnp.float32)
        lane = lax.iota(jnp.int32, 16)

        @pl.loop(0, NPASS)
        def _(p):
            base = p * PASS_SPAN + c * BIN

            @pl.loop(0, CH)
            def _(r):
                @pl.loop(0, D // 16)
                def _(q):
                    grows[r, pl.ds(q * 16, 16)] = zero16

            @pl.loop(0, NSUB // CH)
            def _(z):
                pltpu.sync_copy(grows, sp_agg.at[pl.ds(s * NSUB + z * CH, CH)])

            plsc.subcore_barrier()

            @pl.loop(0, NEC)
            def _(ec):
                eoff = s * SLICE + ec * EC
                pltpu.sync_copy(src_hbm.at[pl.ds(eoff, EC)], src_c)
                pltpu.sync_copy(dst_hbm.at[pl.ds(eoff, EC)], dst_c)

                def filt(i, cur):
                    dvec = dst_c[pl.ds(i * 16, 16)]
                    local = dvec - base
                    m = jnp.logical_and(local >= 0, local < BIN)
                    mi = m.astype(jnp.int32)
                    pos = cur + jnp.cumsum(mi) - 1
                    pr = lax.shift_right_logical(pos, 7)
                    pc = lax.bitwise_and(pos, 127)
                    plsc.store_scatter(sel_dst, [pr, pc], local, mask=m)
                    svec = src_c[pl.ds(i * 16, 16)]
                    plsc.store_scatter(sel_src, [pr, pc], svec, mask=m)
                    return cur + jnp.sum(mi)

                n = lax.fori_loop(0, EC // 16, filt, jnp.int32(0))

                nch = (n + CH - 1) // CH
                npad = (nch * CH - n + 15) // 16

                @pl.loop(0, npad)
                def _(q):
                    pos = n + q * 16 + lane
                    pr = lax.shift_right_logical(pos, 7)
                    pc = lax.bitwise_and(pos, 127)
                    plsc.store_scatter(
                        sel_dst, [pr, pc], jnp.full((16,), DUMMY, jnp.int32))
                    plsc.store_scatter(
                        sel_src, [pr, pc], jnp.zeros((16,), jnp.int32))

                @pl.loop(0, nch)
                def _(j):
                    pltpu.sync_copy(feat_hbm.at[sel_src.at[j]], grows)
                    pltpu.sync_copy(grows, sp_agg.at[sel_dst.at[j]], add=True)

            plsc.subcore_barrier()

            pltpu.sync_copy(sp_agg.at[pl.ds(s * NSUB, NSUB)],
                            agg_hbm.at[pl.ds(base + s * NSUB, NSUB)])

    return k(feat, src, dst)


def _sc_degrees(dst2d_f, dst2d_r, dst2d_rb, zrows):
    mesh = plsc.VectorSubcoreMesh(core_axis_name="c", subcore_axis_name="s")

    @functools.partial(
        pl.kernel,
        out_type=(
            jax.ShapeDtypeStruct((NC * DEG_ROWS,), jnp.float32),
            jax.ShapeDtypeStruct((NC * DEG_ROWS,), jnp.float32),
            jax.ShapeDtypeStruct((NC * DEG_ROWS,), jnp.float32),
        ),
        mesh=mesh,
        scratch_types=[
            pltpu.VMEM((DCH, CH), jnp.int32),
            pltpu.VMEM((CH,), jnp.float32),
            pltpu.VMEM_SHARED((DEG_ROWS,), jnp.float32),
        ],
        compiler_params=_SC_PARAMS,
    )
    def k(df_hbm, dr_hbm, drb_hbm, z_hbm, of_hbm, or_hbm, orb_hbm,
          dstc, ones_v, sp_deg):
        c = lax.axis_index("c")
        s = lax.axis_index("s")
        one16 = jnp.ones((16,), jnp.float32)

        @pl.loop(0, CH // 16)
        def _(r):
            ones_v[pl.ds(r * 16, 16)] = one16

        def one_round(dst_hbm, deg_hbm):
            pltpu.sync_copy(z_hbm, sp_deg.at[pl.ds(s * DSUB, DSUB)])
            plsc.subcore_barrier()
            pltpu.sync_copy(
                dst_hbm.at[pl.ds((c * NS + s) * DCH, DCH)], dstc)

            @pl.loop(0, DCH)
            def _(r):
                pltpu.sync_copy(ones_v, sp_deg.at[dstc.at[r]], add=True)

            plsc.subcore_barrier()
            pltpu.sync_copy(
                sp_deg.at[pl.ds(s * DSUB, DSUB)],
                deg_hbm.at[pl.ds(c * DEG_ROWS + s * DSUB, DSUB)])

        one_round(df_hbm, of_hbm)
        one_round(dr_hbm, or_hbm)
        one_round(drb_hbm, orb_hbm)

    return k(dst2d_f, dst2d_r, dst2d_rb, zrows)


def _tc_combine(agg_f, deg_f, agg_rb, deg_rb, agg_r, deg_r,
                w_f, b_f, w_rb, b_rb, w_r, b_r):
    nb = 50
    rows = N_NODES // nb

    def body(af, df0, df1, arb, drb0, drb1, ar, dr0, dr1,
             wf, bf, wrb, brb, wr, br, hu, hi):
        df = df0[...][0] + df1[...][0]
        drb = drb0[...][0] + drb1[...][0]
        dr = dr0[...][0] + dr1[...][0]
        t = jnp.dot(af[...], wf[...], preferred_element_type=jnp.float32)
        t += df * bf[...]
        t += jnp.dot(arb[...], wrb[...], preferred_element_type=jnp.float32)
        t += drb * brb[...]
        hu[...] = jnp.maximum(t, 0.0)
        u = jnp.dot(ar[...], wr[...], preferred_element_type=jnp.float32)
        u += dr * br[...]
        hi[...] = jnp.maximum(u, 0.0)

    blk = lambda i: (i, 0)
    rep = lambda i: (0, 0)
    dg0 = lambda i: (0, i, 0)
    dg1 = lambda i: (1, i, 0)
    return pl.pallas_call(
        body,
        grid=(nb,),
        in_specs=[
            pl.BlockSpec((rows, D), blk),
            pl.BlockSpec((1, rows, 1), dg0),
            pl.BlockSpec((1, rows, 1), dg1),
            pl.BlockSpec((rows, D), blk),
            pl.BlockSpec((1, rows, 1), dg0),
            pl.BlockSpec((1, rows, 1), dg1),
            pl.BlockSpec((rows, D), blk),
            pl.BlockSpec((1, rows, 1), dg0),
            pl.BlockSpec((1, rows, 1), dg1),
            pl.BlockSpec((D, D), rep),
            pl.BlockSpec((1, D), rep),
            pl.BlockSpec((D, D), rep),
            pl.BlockSpec((1, D), rep),
            pl.BlockSpec((D, D), rep),
            pl.BlockSpec((1, D), rep),
        ],
        out_specs=[
            pl.BlockSpec((rows, D), blk),
            pl.BlockSpec((rows, D), blk),
        ],
        out_shape=[
            jax.ShapeDtypeStruct((N_NODES, D), jnp.float32),
            jax.ShapeDtypeStruct((N_NODES, D), jnp.float32),
        ],
    )(agg_f, deg_f, deg_f, agg_rb, deg_rb, deg_rb, agg_r, deg_r, deg_r,
      w_f, b_f, w_rb, b_rb, w_r, b_r)


def _pad_edges(edge_index):
    src = jnp.pad(edge_index[0], (0, EP - E_EDGES))
    dst = jnp.pad(edge_index[1], (0, EP - E_EDGES), constant_values=PAD_DST)
    dst2d = jnp.pad(edge_index[1], (0, EP_DEG - E_EDGES),
                    constant_values=N_NODES).reshape(EP_DEG // CH, CH)
    return src, dst, dst2d


def kernel(feat_user, feat_item, edge_index_follows, edge_index_rates,
           edge_index_rated_by, W_follows, b_follows, W_rates, b_rates,
           W_rated_by, b_rated_by):
    src_f, dst_f, d2_f = _pad_edges(edge_index_follows)
    src_r, dst_r, d2_r = _pad_edges(edge_index_rates)
    src_rb, dst_rb, d2_rb = _pad_edges(edge_index_rated_by)
    zrows = jnp.zeros((DSUB,), jnp.float32)

    deg_f, deg_r, deg_rb = _sc_degrees(d2_f, d2_r, d2_rb, zrows)
    deg_f = deg_f.reshape(NC, DEG_ROWS)[:, :N_NODES, None]
    deg_r = deg_r.reshape(NC, DEG_ROWS)[:, :N_NODES, None]
    deg_rb = deg_rb.reshape(NC, DEG_ROWS)[:, :N_NODES, None]
    agg_f = _sc_aggregate(feat_user, src_f, dst_f)
    agg_rb = _sc_aggregate(feat_item, src_rb, dst_rb)
    agg_r = _sc_aggregate(feat_user, src_r, dst_r)

    h_user, h_item = _tc_combine(
        agg_f, deg_f, agg_rb, deg_rb, agg_r, deg_r,
        W_follows, b_follows.reshape(1, D),
        W_rated_by, b_rated_by.reshape(1, D),
        W_rates, b_rates.reshape(1, D),
    )
    return (h_user, h_item)
